# baseline (speedup 1.0000x reference)
import functools

import jax
import jax.numpy as jnp
import numpy as np
from jax import lax
from jax.experimental import pallas as pl
from jax.experimental.pallas import tpu as pltpu
from jax.experimental.pallas import tpu_sc as plsc

N = 10000
E = 320000
H = 512
EF = 30
G = 64
NC = 4
CW = 128
NT = 16
EPT = E // NT
BE = 80
NW = EPT // BE
E2 = E + 2 * BE
ZR = 16
DR = 10240

_SC_PARAMS = pltpu.CompilerParams(use_tc_tiling_on_sc=False,
                                  needs_layout_passes=False)

_P128 = np.empty((128,), np.int32)
for _j in range(4):
    for _i in range(16):
        _P128[32 * _j + 2 * _i] = 32 * _j + _i
        _P128[32 * _j + 2 * _i + 1] = 32 * _j + _i + 16
_PERM = np.concatenate([c * CW + _P128 for c in range(NC)])


_BEA = 8000


def _eaw_body(ea_ref, w_ref, out_ref):
    ea = ea_ref[...]
    for c in range(NC):
        out_ref[0, c] = jnp.dot(
            ea, w_ref[0, c],
            preferred_element_type=jnp.float32).astype(jnp.bfloat16)


def _eaw_kernel(edge_attr, w1bs):
    nlayer = w1bs.shape[0]
    return pl.pallas_call(
        _eaw_body,
        grid=(E // _BEA, nlayer),
        in_specs=[
            pl.BlockSpec((_BEA, EF), lambda e, l: (e, 0)),
            pl.BlockSpec((1, NC, EF, CW), lambda e, l: (l, 0, 0, 0)),
        ],
        out_specs=pl.BlockSpec((1, NC, _BEA, CW), lambda e, l: (l, 0, e, 0)),
        out_shape=jax.ShapeDtypeStruct((nlayer, NC, E, CW), jnp.bfloat16),
    )(edge_attr, w1bs)


_NBLK = 2000


def _proj0_body(x_ref, w_ref, b_ref, out_ref):
    hw = (jnp.dot(x_ref[...], w_ref[...],
                  preferred_element_type=jnp.float32)
          + b_ref[...]).astype(jnp.bfloat16)
    for c in range(NC):
        out_ref[c] = hw[:, c * CW:(c + 1) * CW]


def _proj0(x, w, b):
    cur = x.shape[1]
    return pl.pallas_call(
        _proj0_body,
        grid=(N // _NBLK,),
        in_specs=[
            pl.BlockSpec((_NBLK, cur), lambda i: (i, 0)),
            pl.BlockSpec((cur, H), lambda i: (0, 0)),
            pl.BlockSpec((1, H), lambda i: (0, 0)),
        ],
        out_specs=pl.BlockSpec((NC, _NBLK, CW), lambda i: (0, i, 0)),
        out_shape=jax.ShapeDtypeStruct((NC, N, CW), jnp.bfloat16),
    )(x, w, b)


def _sc_layer_body(hw_hbm, eaw_hbm, gidx_hbm, dst_hbm, agg_hbm,
                   gib0, gib1, db0, db1, db2, db3,
                   gb0, gb1, eb0, eb1, mb0, mb1, zbuf, acc_sh,
                   isem0, isem1, gsem0, gsem1, esem0, esem1,
                   ssem0, ssem1, ssem2, ssem3):
    cid = lax.axis_index("c")
    sid = lax.axis_index("s")
    gib = (gib0, gib1)
    db = (db0, db1, db2, db3)
    gb = (gb0, gb1)
    eb = (eb0, eb1)
    mb = (mb0, mb1)
    isem = (isem0, isem1)
    gsem = (gsem0, gsem1)
    esem = (esem0, esem1)
    ssem = (ssem0, ssem1, ssem2, ssem3)

    def zfill(i, c):
        for k in range(CW // 16):
            zbuf[i, pl.ds(k * 16, 16)] = jnp.zeros((16,), jnp.float32)
        return c
    lax.fori_loop(0, ZR, zfill, 0)

    for p in range(2):
        chunk = cid + 2 * p
        ibase = chunk * E2 + sid * EPT
        abase = chunk * E + sid * EPT
        dbase = sid * EPT

        @pl.when(sid < NT - 1)
        def _():
            def zc(j, c):
                pltpu.sync_copy(zbuf, acc_sh.at[pl.ds(sid * 640 + j * ZR,
                                                      ZR)])
                return c
            lax.fori_loop(0, 640 // ZR, zc, 0)

        @pl.when(sid == NT - 1)
        def _():
            def zc(j, c):
                pltpu.sync_copy(zbuf, acc_sh.at[pl.ds(9600 + j * ZR, ZR)])
                return c
            lax.fori_loop(0, 400 // ZR, zc, 0)
        plsc.subcore_barrier()

        def issue_idx(w, k2, k4):
            pltpu.async_copy(gidx_hbm.at[pl.ds(ibase + w * BE, BE)],
                             gib[k2], isem[k2])
            pltpu.async_copy(dst_hbm.at[pl.ds(dbase + w * BE, BE)],
                             db[k4], isem[k2])

        def wait_idx(w, k2, k4):
            pltpu.make_async_copy(gidx_hbm.at[pl.ds(ibase + w * BE, BE)],
                                  gib[k2], isem[k2]).wait()
            pltpu.make_async_copy(dst_hbm.at[pl.ds(dbase + w * BE, BE)],
                                  db[k4], isem[k2]).wait()

        def issue_ge(w, k2):
            pltpu.async_copy(hw_hbm.at[gib[k2]], gb[k2], gsem[k2])
            pltpu.async_copy(eaw_hbm.at[pl.ds(abase + w * BE, BE)],
                             eb[k2], esem[k2])

        def wait_ge(w, k2):
            pltpu.make_async_copy(hw_hbm.at[gib[k2]], gb[k2],
                                  gsem[k2]).wait()
            pltpu.make_async_copy(eaw_hbm.at[pl.ds(abase + w * BE, BE)],
                                  eb[k2], esem[k2]).wait()

        def compute(k2):
            def rowfn(i, c):
                for u in range(2):
                    r = 2 * i + u
                    for j in range(CW // 32):
                        s32 = pl.ds(32 * j, 32)
                        m = jnp.maximum(gb[k2][r, s32] + eb[k2][r, s32],
                                        jnp.bfloat16(0.0))
                        m0, m1 = plsc.unpack(
                            m, format=plsc.PackFormat.INTERLEAVED,
                            preferred_element_type=jnp.float32)
                        mb[k2][r, pl.ds(32 * j, 16)] = m0
                        mb[k2][r, pl.ds(32 * j + 16, 16)] = m1
                return c
            lax.fori_loop(0, BE // 2, rowfn, 0)

        def issue_scat(k2, k4):
            pltpu.async_copy(mb[k2], acc_sh.at[db[k4]], ssem[k4], add=True)

        def wait_scat(k2, k4):
            pltpu.make_async_copy(mb[k2], acc_sh.at[db[k4]],
                                  ssem[k4]).wait()

        issue_idx(0, 0, 0)
        issue_idx(1, 1, 1)
        wait_idx(0, 0, 0)
        issue_ge(0, 0)
        wait_ge(0, 0)
        compute(0)
        issue_scat(0, 0)
        wait_idx(1, 1, 1)
        issue_ge(1, 1)
        issue_idx(2, 0, 2)

        def step(i, c):
            for b in range(4):
                w = 1 + i * 4 + b
                k2 = (1 + b) % 2
                k4 = (1 + b) % 4
                wait_ge(w, k2)
                compute(k2)
                issue_scat(k2, k4)
                wait_idx(w + 1, (k2 + 1) % 2, (k4 + 1) % 4)
                wait_scat((k2 + 1) % 2, (k4 + 3) % 4)
                issue_ge(w + 1, (k2 + 1) % 2)
                issue_idx(w + 2, k2, (k4 + 2) % 4)
            return c
        lax.fori_loop(0, (NW - 2) // 4, step, 0)

        wait_ge(NW - 1, (NW - 1) % 2)
        compute((NW - 1) % 2)
        wait_scat((NW - 2) % 2, (NW - 2) % 4)
        pltpu.sync_copy(mb[(NW - 1) % 2], acc_sh.at[db[(NW - 1) % 4]],
                        add=True)
        wait_idx(NW, NW % 2, NW % 4)
        plsc.subcore_barrier()

        @pl.when(sid < NT - 1)
        def _():
            pltpu.sync_copy(acc_sh.at[pl.ds(sid * 640, 640)],
                            agg_hbm.at[pl.ds(chunk * N + sid * 640, 640)])

        @pl.when(sid == NT - 1)
        def _():
            pltpu.sync_copy(acc_sh.at[pl.ds(9600, 400)],
                            agg_hbm.at[pl.ds(chunk * N + 9600, 400)])
        plsc.subcore_barrier()


def _sc_layer(hw_flat, eaw_flat, gidx, dstp):
    scratch = [
        pltpu.VMEM((BE,), jnp.int32),
        pltpu.VMEM((BE,), jnp.int32),
        pltpu.VMEM((BE,), jnp.int32),
        pltpu.VMEM((BE,), jnp.int32),
        pltpu.VMEM((BE,), jnp.int32),
        pltpu.VMEM((BE,), jnp.int32),
        pltpu.VMEM((BE, CW), jnp.bfloat16),
        pltpu.VMEM((BE, CW), jnp.bfloat16),
        pltpu.VMEM((BE, CW), jnp.bfloat16),
        pltpu.VMEM((BE, CW), jnp.bfloat16),
        pltpu.VMEM((BE, CW), jnp.float32),
        pltpu.VMEM((BE, CW), jnp.float32),
        pltpu.VMEM((ZR, CW), jnp.float32),
        pltpu.VMEM_SHARED((N, CW), jnp.float32),
    ] + [pltpu.SemaphoreType.DMA] * 10
    fn = pl.kernel(
        _sc_layer_body,
        out_type=jax.ShapeDtypeStruct((NC * N, CW), jnp.float32),
        mesh=plsc.VectorSubcoreMesh(core_axis_name="c", subcore_axis_name="s"),
        scratch_types=scratch,
        compiler_params=_SC_PARAMS,
    )
    return fn(hw_flat, eaw_flat, gidx, dstp)


def _gidx_body(s_ref, out_ref):
    sv = s_ref[...]
    for c in range(NC):
        out_ref[c] = sv + c * N


def _gidx(src2d):
    r, q = src2d.shape
    return pl.pallas_call(
        _gidx_body,
        out_shape=jax.ShapeDtypeStruct((NC, r, q), jnp.int32),
    )(src2d)


_EPW = E // 32
_DNF = _EPW // BE


def _sc_deg_body(dst_hbm, deg_hbm, dbuf, ones_buf, dzbuf, dacc_sh):
    cid = lax.axis_index("c")
    sid = lax.axis_index("s")

    def ofill(i, c):
        ones_buf[i, pl.ds(0, 16)] = jnp.ones((16,), jnp.float32)
        return c
    lax.fori_loop(0, BE, ofill, 0)

    def dzfill(i, c):
        dzbuf[i, pl.ds(0, 16)] = jnp.zeros((16,), jnp.float32)
        return c
    lax.fori_loop(0, DR // NT, dzfill, 0)

    pltpu.sync_copy(dzbuf, dacc_sh.at[pl.ds(sid * (DR // NT), DR // NT)])
    plsc.subcore_barrier()

    base = cid * (E // 2) + sid * _EPW

    def window(b, c):
        pltpu.sync_copy(dst_hbm.at[pl.ds(base + b * BE, BE)], dbuf)
        pltpu.sync_copy(ones_buf, dacc_sh.at[dbuf], add=True)
        return c
    lax.fori_loop(0, _DNF, window, 0)
    plsc.subcore_barrier()

    @pl.when(sid < NT - 1)
    def _():
        pltpu.sync_copy(dacc_sh.at[pl.ds(sid * 640, 640)],
                        deg_hbm.at[pl.ds(cid * N + sid * 640, 640)])

    @pl.when(sid == NT - 1)
    def _():
        pltpu.sync_copy(dacc_sh.at[pl.ds(9600, 400)],
                        deg_hbm.at[pl.ds(cid * N + 9600, 400)])


def _sc_deg(dst):
    fn = pl.kernel(
        _sc_deg_body,
        out_type=jax.ShapeDtypeStruct((2 * N, 16), jnp.float32),
        mesh=plsc.VectorSubcoreMesh(core_axis_name="c", subcore_axis_name="s"),
        scratch_types=[
            pltpu.VMEM((BE,), jnp.int32),
            pltpu.VMEM((BE, 16), jnp.float32),
            pltpu.VMEM((DR // NT, 16), jnp.float32),
            pltpu.VMEM_SHARED((DR, 16), jnp.float32),
        ],
        compiler_params=_SC_PARAMS,
    )
    return fn(dst)


def _dense_body_maker(l, has_next):
    def body(*args):
        if has_next:
            (h_ref, agg_ref, deg_ref, wut, w2u, b2u, bu, nw, fw, fb,
             w1n, b1n, hn_ref, hwn_ref) = args
        else:
            (h_ref, agg_ref, deg_ref, wut, w2u, b2u, bu, nw, fw, fb,
             hn_ref) = args
        hb = h_ref[...]
        aggf = jnp.concatenate([agg_ref[c] for c in range(NC)], axis=-1)
        degv = deg_ref[0, :, :1] + deg_ref[1, :, :1]
        pre = (jnp.dot(hb, wut[...], preferred_element_type=jnp.float32)
               + jnp.dot(aggf, w2u[...], preferred_element_type=jnp.float32)
               + degv * b2u[...] + bu[...])
        xn = jnp.maximum(pre, 0.0)
        ms = jnp.mean(xn * xn, axis=-1, keepdims=True)
        xn = xn * lax.rsqrt(ms + 1e-6) * nw[...]
        y = jnp.dot(xn, fw[...], preferred_element_type=jnp.float32) + fb[...]
        hn = hb + y if l > 0 else y
        hn_ref[...] = hn
        if has_next:
            hw = (jnp.dot(hn, w1n[...], preferred_element_type=jnp.float32)
                  + b1n[...]).astype(jnp.bfloat16)
            for c in range(NC):
                hwn_ref[c] = hw[:, c * CW:(c + 1) * CW]
    return body


def _dense(l, h, agg4, deg16, wut, w2u, b2u, bu, nw, fw, fb, w1n, b1n):
    cur = h.shape[1]
    has_next = w1n is not None
    in_specs = [
        pl.BlockSpec((_NBLK, cur), lambda i: (i, 0)),
        pl.BlockSpec((NC, _NBLK, CW), lambda i: (0, i, 0)),
        pl.BlockSpec((2, _NBLK, 16), lambda i: (0, i, 0)),
        pl.BlockSpec((cur, H), lambda i: (0, 0)),
        pl.BlockSpec((H, H), lambda i: (0, 0)),
        pl.BlockSpec((1, H), lambda i: (0, 0)),
        pl.BlockSpec((1, H), lambda i: (0, 0)),
        pl.BlockSpec((1, H), lambda i: (0, 0)),
        pl.BlockSpec((H, H), lambda i: (0, 0)),
        pl.BlockSpec((1, H), lambda i: (0, 0)),
    ]
    args = [h, agg4, deg16, wut, w2u, b2u, bu, nw, fw, fb]
    out_specs = [pl.BlockSpec((_NBLK, H), lambda i: (i, 0))]
    out_shape = [jax.ShapeDtypeStruct((N, H), jnp.float32)]
    if has_next:
        in_specs.append(pl.BlockSpec((H, H), lambda i: (0, 0)))
        in_specs.append(pl.BlockSpec((1, H), lambda i: (0, 0)))
        args.append(w1n)
        args.append(b1n)
        out_specs.append(pl.BlockSpec((NC, _NBLK, CW), lambda i: (0, i, 0)))
        out_shape.append(jax.ShapeDtypeStruct((NC, N, CW), jnp.bfloat16))
    res = pl.pallas_call(
        _dense_body_maker(l, has_next),
        grid=(N // _NBLK,),
        in_specs=in_specs,
        out_specs=out_specs,
        out_shape=out_shape,
    )(*args)
    return res if has_next else (res[0], None)


def _readout_body(h_ref, batch_ref, gw, gb, aw, p1w, p1b, pn, p2w, p2b,
                  out_ref):
    h = h_ref[...]
    z = jnp.dot(h, gw[...], preferred_element_type=jnp.float32) + gb[...]
    awr = aw[...]
    att = jnp.sum(z * awr, axis=1, keepdims=True)
    cg = jnp.sum(gb[...] * awr)
    b2 = (batch_ref[...] == lax.broadcasted_iota(jnp.int32, (N, G), 1))
    b2 = b2.astype(jnp.float32)
    counts = jnp.sum(b2, axis=0, keepdims=True)
    m = jnp.max(counts)
    neg = jnp.float32(-1e30)
    segmax = jnp.max(jnp.where(b2 > 0, att, neg), axis=0, keepdims=True)
    ghost = jnp.where(counts < m, cg, neg)
    maxg = jnp.maximum(segmax, ghost)
    maxpn = jnp.sum(b2 * maxg, axis=1, keepdims=True)
    w = jnp.exp(att - maxpn)
    denom = (jnp.sum(b2 * w, axis=0, keepdims=True)
             + (m - counts) * jnp.exp(ghost - maxg))
    denom_pn = jnp.sum(b2 * denom, axis=1, keepdims=True)
    bz = b2 * (w / denom_pn)
    hg = lax.dot_general(bz, z, (((0,), (0,)), ((), ())),
                         preferred_element_type=jnp.float32)
    zg = jnp.dot(hg, p1w[...], preferred_element_type=jnp.float32) + p1b[...]
    zg = zg * lax.rsqrt(jnp.mean(zg * zg, axis=-1, keepdims=True) + 1e-6)
    zg = zg * pn[...]
    zg = jnp.maximum(zg, 0.0)
    out_ref[...] = jnp.dot(zg, p2w[...],
                           preferred_element_type=jnp.float32) + p2b[...]


def _readout(h, batch2d, gw, gb, aw, p1w, p1b, pn, p2w, p2b):
    pdim = p2w.shape[1]
    return pl.pallas_call(
        _readout_body,
        out_shape=jax.ShapeDtypeStruct((G, pdim), jnp.float32),
        compiler_params=pltpu.CompilerParams(
            vmem_limit_bytes=100 * 1024 * 1024),
    )(h, batch2d, gw, gb, aw, p1w, p1b, pn, p2w, p2b)


def kernel(x, edge_index, edge_attr, batch, params):
    src = edge_index[0]
    dst = edge_index[1]
    convs = params["convs"]
    ffns = params["ffns"]
    norms = params["norms"]
    nlayer = len(ffns)

    w1a, b1, w1b, wut, w2u, b2u, bu = [], [], [], [], [], [], []
    for l in range(nlayer):
        cur = x.shape[1] if l == 0 else H
        m1w = convs[l]["m1"]["w"]
        w1a.append(m1w[:cur][:, _PERM])
        b1.append(convs[l]["m1"]["b"][_PERM].reshape(1, H))
        w1b.append(m1w[cur:cur + EF][:, _PERM]
                   .reshape(EF, NC, CW).transpose(1, 0, 2))
        uw = convs[l]["u"]["w"]
        wub = uw[cur:]
        wut.append(uw[:cur])
        w2u.append(convs[l]["m2"]["w"] @ wub)
        b2u.append((convs[l]["m2"]["b"] @ wub).reshape(1, H))
        bu.append(convs[l]["u"]["b"].reshape(1, H))
    w1bs = jnp.stack(w1b)

    eaw = _eaw_kernel(edge_attr, w1bs)
    hw = _proj0(x, w1a[0], b1[0])

    srcp = jnp.concatenate([src, jnp.zeros((2 * BE,), src.dtype)])
    dstp = jnp.concatenate([dst, jnp.zeros((2 * BE,), dst.dtype)])
    gidx = _gidx(srcp.reshape(40, E2 // 40)).reshape(NC * E2)

    h = x
    deg16 = _sc_deg(dst).reshape(2, N, 16)
    for l in range(nlayer):
        eaw_l = eaw[l].reshape(NC * E, CW)
        agg = _sc_layer(hw.reshape(NC * N, CW), eaw_l, gidx, dstp)
        agg4 = agg.reshape(NC, N, CW)
        last = l == nlayer - 1
        h, hwn = _dense(
            l, h, agg4, deg16,
            wut[l], w2u[l], b2u[l], bu[l],
            norms[l].reshape(1, H),
            ffns[l]["w"], ffns[l]["b"].reshape(1, H),
            None if last else w1a[l + 1],
            None if last else b1[l + 1],
        )
        if not last:
            hw = hwn

    return _readout(
        h, batch.reshape(N, 1),
        params["gap"]["w"], params["gap"]["b"].reshape(1, H),
        params["att"]["w"].reshape(1, H),
        params["p1"]["w"], params["p1"]["b"].reshape(1, -1),
        params["pn"].reshape(1, -1),
        params["p2"]["w"], params["p2"]["b"].reshape(1, -1),
    )


# speedup vs baseline: 1.4195x; 1.4195x over previous
import functools

import jax
import jax.numpy as jnp
import numpy as np
from jax import lax
from jax.experimental import pallas as pl
from jax.experimental.pallas import tpu as pltpu
from jax.experimental.pallas import tpu_sc as plsc

N = 10000
E = 320000
H = 512
EF = 30
G = 64
NC = 4
CW = 128
NT = 16
EPT = E // NT
BE = 80
NW = EPT // BE
E2 = E + 2 * BE
ZR = 16
DR = 10240

_SC_PARAMS = pltpu.CompilerParams(use_tc_tiling_on_sc=False)


_BEA = 8000


def _eaw_body(ea_ref, w_ref, out_ref):
    ea = ea_ref[...]
    for c in range(NC):
        out_ref[0, c] = jnp.dot(ea, w_ref[0, c],
                                preferred_element_type=jnp.float32)


def _eaw_kernel(edge_attr, w1bs):
    nlayer = w1bs.shape[0]
    return pl.pallas_call(
        _eaw_body,
        grid=(E // _BEA, nlayer),
        in_specs=[
            pl.BlockSpec((_BEA, EF), lambda e, l: (e, 0)),
            pl.BlockSpec((1, NC, EF, CW), lambda e, l: (l, 0, 0, 0)),
        ],
        out_specs=pl.BlockSpec((1, NC, _BEA, CW), lambda e, l: (l, 0, e, 0)),
        out_shape=jax.ShapeDtypeStruct((nlayer, NC, E, CW), jnp.float32),
    )(edge_attr, w1bs)


_NBLK = 2000


def _proj0_body(x_ref, w_ref, b_ref, out_ref):
    hw = jnp.dot(x_ref[...], w_ref[...],
                 preferred_element_type=jnp.float32) + b_ref[...]
    for c in range(NC):
        out_ref[c] = hw[:, c * CW:(c + 1) * CW]


def _proj0(x, w, b):
    cur = x.shape[1]
    return pl.pallas_call(
        _proj0_body,
        grid=(N // _NBLK,),
        in_specs=[
            pl.BlockSpec((_NBLK, cur), lambda i: (i, 0)),
            pl.BlockSpec((cur, H), lambda i: (0, 0)),
            pl.BlockSpec((1, H), lambda i: (0, 0)),
        ],
        out_specs=pl.BlockSpec((NC, _NBLK, CW), lambda i: (0, i, 0)),
        out_shape=jax.ShapeDtypeStruct((NC, N, CW), jnp.float32),
    )(x, w, b)


def _sc_layer_body(hw_hbm, eaw_hbm, gidx_hbm, dst_hbm, agg_hbm,
                   gib0, gib1, db0, db1, db2, db3,
                   gb0, gb1, eb0, eb1, zbuf, acc_sh,
                   isem0, isem1, gsem0, gsem1, esem0, esem1,
                   ssem0, ssem1, ssem2, ssem3):
    cid = lax.axis_index("c")
    sid = lax.axis_index("s")
    gib = (gib0, gib1)
    db = (db0, db1, db2, db3)
    gb = (gb0, gb1)
    eb = (eb0, eb1)
    isem = (isem0, isem1)
    gsem = (gsem0, gsem1)
    esem = (esem0, esem1)
    ssem = (ssem0, ssem1, ssem2, ssem3)

    def zfill(i, c):
        for k in range(CW // 16):
            zbuf[i, pl.ds(k * 16, 16)] = jnp.zeros((16,), jnp.float32)
        return c
    lax.fori_loop(0, ZR, zfill, 0)

    for p in range(2):
        chunk = cid + 2 * p
        ibase = chunk * E2 + sid * EPT
        abase = chunk * E + sid * EPT
        dbase = sid * EPT

        @pl.when(sid < NT - 1)
        def _():
            def zc(j, c):
                pltpu.sync_copy(zbuf, acc_sh.at[pl.ds(sid * 640 + j * ZR,
                                                      ZR)])
                return c
            lax.fori_loop(0, 640 // ZR, zc, 0)

        @pl.when(sid == NT - 1)
        def _():
            def zc(j, c):
                pltpu.sync_copy(zbuf, acc_sh.at[pl.ds(9600 + j * ZR, ZR)])
                return c
            lax.fori_loop(0, 400 // ZR, zc, 0)
        plsc.subcore_barrier()

        def issue_idx(w, k2, k4):
            pltpu.async_copy(gidx_hbm.at[pl.ds(ibase + w * BE, BE)],
                             gib[k2], isem[k2])
            pltpu.async_copy(dst_hbm.at[pl.ds(dbase + w * BE, BE)],
                             db[k4], isem[k2])

        def wait_idx(w, k2, k4):
            pltpu.make_async_copy(gidx_hbm.at[pl.ds(ibase + w * BE, BE)],
                                  gib[k2], isem[k2]).wait()
            pltpu.make_async_copy(dst_hbm.at[pl.ds(dbase + w * BE, BE)],
                                  db[k4], isem[k2]).wait()

        def issue_ge(w, k2):
            pltpu.async_copy(hw_hbm.at[gib[k2]], gb[k2], gsem[k2])
            pltpu.async_copy(eaw_hbm.at[pl.ds(abase + w * BE, BE)],
                             eb[k2], esem[k2])

        def wait_ge(w, k2):
            pltpu.make_async_copy(hw_hbm.at[gib[k2]], gb[k2],
                                  gsem[k2]).wait()
            pltpu.make_async_copy(eaw_hbm.at[pl.ds(abase + w * BE, BE)],
                                  eb[k2], esem[k2]).wait()

        def compute(k2):
            def rowfn(r, c):
                for j in range(CW // 16):
                    s = pl.ds(j * 16, 16)
                    gb[k2][r, s] = jnp.maximum(gb[k2][r, s] + eb[k2][r, s],
                                               0.0)
                return c
            lax.fori_loop(0, BE, rowfn, 0)

        def issue_scat(k2, k4):
            pltpu.async_copy(gb[k2], acc_sh.at[db[k4]], ssem[k4], add=True)

        def wait_scat(k2, k4):
            pltpu.make_async_copy(gb[k2], acc_sh.at[db[k4]],
                                  ssem[k4]).wait()

        issue_idx(0, 0, 0)
        issue_idx(1, 1, 1)
        wait_idx(0, 0, 0)
        issue_ge(0, 0)
        wait_ge(0, 0)
        compute(0)
        issue_scat(0, 0)
        wait_idx(1, 1, 1)
        issue_ge(1, 1)
        issue_idx(2, 0, 2)

        def step(i, c):
            for b in range(4):
                w = 1 + i * 4 + b
                k2 = (1 + b) % 2
                k4 = (1 + b) % 4
                wait_ge(w, k2)
                compute(k2)
                issue_scat(k2, k4)
                wait_idx(w + 1, (k2 + 1) % 2, (k4 + 1) % 4)
                wait_scat((k2 + 1) % 2, (k4 + 3) % 4)
                issue_ge(w + 1, (k2 + 1) % 2)
                issue_idx(w + 2, k2, (k4 + 2) % 4)
            return c
        lax.fori_loop(0, (NW - 2) // 4, step, 0)

        wait_ge(NW - 1, (NW - 1) % 2)
        compute((NW - 1) % 2)
        wait_scat((NW - 2) % 2, (NW - 2) % 4)
        pltpu.sync_copy(gb[(NW - 1) % 2], acc_sh.at[db[(NW - 1) % 4]],
                        add=True)
        wait_idx(NW, NW % 2, NW % 4)
        plsc.subcore_barrier()

        @pl.when(sid < NT - 1)
        def _():
            pltpu.sync_copy(acc_sh.at[pl.ds(sid * 640, 640)],
                            agg_hbm.at[pl.ds(chunk * N + sid * 640, 640)])

        @pl.when(sid == NT - 1)
        def _():
            pltpu.sync_copy(acc_sh.at[pl.ds(9600, 400)],
                            agg_hbm.at[pl.ds(chunk * N + 9600, 400)])
        plsc.subcore_barrier()


def _sc_layer(hw_flat, eaw_flat, gidx, dstp):
    scratch = [
        pltpu.VMEM((BE,), jnp.int32),
        pltpu.VMEM((BE,), jnp.int32),
        pltpu.VMEM((BE,), jnp.int32),
        pltpu.VMEM((BE,), jnp.int32),
        pltpu.VMEM((BE,), jnp.int32),
        pltpu.VMEM((BE,), jnp.int32),
        pltpu.VMEM((BE, CW), jnp.float32),
        pltpu.VMEM((BE, CW), jnp.float32),
        pltpu.VMEM((BE, CW), jnp.float32),
        pltpu.VMEM((BE, CW), jnp.float32),
        pltpu.VMEM((ZR, CW), jnp.float32),
        pltpu.VMEM_SHARED((N, CW), jnp.float32),
    ] + [pltpu.SemaphoreType.DMA] * 10
    fn = pl.kernel(
        _sc_layer_body,
        out_type=jax.ShapeDtypeStruct((NC * N, CW), jnp.float32),
        mesh=plsc.VectorSubcoreMesh(core_axis_name="c", subcore_axis_name="s"),
        scratch_types=scratch,
        compiler_params=_SC_PARAMS,
    )
    return fn(hw_flat, eaw_flat, gidx, dstp)


def _gidx_body(s_ref, out_ref):
    sv = s_ref[...]
    for c in range(NC):
        out_ref[c] = sv + c * N


def _gidx(src2d):
    r, q = src2d.shape
    return pl.pallas_call(
        _gidx_body,
        out_shape=jax.ShapeDtypeStruct((NC, r, q), jnp.int32),
    )(src2d)


_EPW = E // 32
_DNF = _EPW // BE


def _sc_deg_body(dst_hbm, deg_hbm, dbuf, ones_buf, dzbuf, dacc_sh):
    cid = lax.axis_index("c")
    sid = lax.axis_index("s")

    def ofill(i, c):
        ones_buf[i, pl.ds(0, 16)] = jnp.ones((16,), jnp.float32)
        return c
    lax.fori_loop(0, BE, ofill, 0)

    def dzfill(i, c):
        dzbuf[i, pl.ds(0, 16)] = jnp.zeros((16,), jnp.float32)
        return c
    lax.fori_loop(0, DR // NT, dzfill, 0)

    pltpu.sync_copy(dzbuf, dacc_sh.at[pl.ds(sid * (DR // NT), DR // NT)])
    plsc.subcore_barrier()

    base = cid * (E // 2) + sid * _EPW

    def window(b, c):
        pltpu.sync_copy(dst_hbm.at[pl.ds(base + b * BE, BE)], dbuf)
        pltpu.sync_copy(ones_buf, dacc_sh.at[dbuf], add=True)
        return c
    lax.fori_loop(0, _DNF, window, 0)
    plsc.subcore_barrier()

    @pl.when(sid < NT - 1)
    def _():
        pltpu.sync_copy(dacc_sh.at[pl.ds(sid * 640, 640)],
                        deg_hbm.at[pl.ds(cid * N + sid * 640, 640)])

    @pl.when(sid == NT - 1)
    def _():
        pltpu.sync_copy(dacc_sh.at[pl.ds(9600, 400)],
                        deg_hbm.at[pl.ds(cid * N + 9600, 400)])


def _sc_deg(dst):
    fn = pl.kernel(
        _sc_deg_body,
        out_type=jax.ShapeDtypeStruct((2 * N, 16), jnp.float32),
        mesh=plsc.VectorSubcoreMesh(core_axis_name="c", subcore_axis_name="s"),
        scratch_types=[
            pltpu.VMEM((BE,), jnp.int32),
            pltpu.VMEM((BE, 16), jnp.float32),
            pltpu.VMEM((DR // NT, 16), jnp.float32),
            pltpu.VMEM_SHARED((DR, 16), jnp.float32),
        ],
        compiler_params=_SC_PARAMS,
    )
    return fn(dst)


def _dense_body_maker(l, has_next):
    def body(*args):
        if has_next:
            (h_ref, agg_ref, deg_ref, wut, w2u, b2u, bu, nw, fw, fb,
             w1n, b1n, hn_ref, hwn_ref) = args
        else:
            (h_ref, agg_ref, deg_ref, wut, w2u, b2u, bu, nw, fw, fb,
             hn_ref) = args
        hb = h_ref[...]
        aggf = jnp.concatenate([agg_ref[c] for c in range(NC)], axis=-1)
        degv = deg_ref[0, :, :1] + deg_ref[1, :, :1]
        pre = (jnp.dot(hb, wut[...], preferred_element_type=jnp.float32)
               + jnp.dot(aggf, w2u[...], preferred_element_type=jnp.float32)
               + degv * b2u[...] + bu[...])
        xn = jnp.maximum(pre, 0.0)
        ms = jnp.mean(xn * xn, axis=-1, keepdims=True)
        xn = xn * lax.rsqrt(ms + 1e-6) * nw[...]
        y = jnp.dot(xn, fw[...], preferred_element_type=jnp.float32) + fb[...]
        hn = hb + y if l > 0 else y
        hn_ref[...] = hn
        if has_next:
            hw = jnp.dot(hn, w1n[...],
                         preferred_element_type=jnp.float32) + b1n[...]
            for c in range(NC):
                hwn_ref[c] = hw[:, c * CW:(c + 1) * CW]
    return body


def _dense(l, h, agg4, deg16, wut, w2u, b2u, bu, nw, fw, fb, w1n, b1n):
    cur = h.shape[1]
    has_next = w1n is not None
    in_specs = [
        pl.BlockSpec((_NBLK, cur), lambda i: (i, 0)),
        pl.BlockSpec((NC, _NBLK, CW), lambda i: (0, i, 0)),
        pl.BlockSpec((2, _NBLK, 16), lambda i: (0, i, 0)),
        pl.BlockSpec((cur, H), lambda i: (0, 0)),
        pl.BlockSpec((H, H), lambda i: (0, 0)),
        pl.BlockSpec((1, H), lambda i: (0, 0)),
        pl.BlockSpec((1, H), lambda i: (0, 0)),
        pl.BlockSpec((1, H), lambda i: (0, 0)),
        pl.BlockSpec((H, H), lambda i: (0, 0)),
        pl.BlockSpec((1, H), lambda i: (0, 0)),
    ]
    args = [h, agg4, deg16, wut, w2u, b2u, bu, nw, fw, fb]
    out_specs = [pl.BlockSpec((_NBLK, H), lambda i: (i, 0))]
    out_shape = [jax.ShapeDtypeStruct((N, H), jnp.float32)]
    if has_next:
        in_specs.append(pl.BlockSpec((H, H), lambda i: (0, 0)))
        in_specs.append(pl.BlockSpec((1, H), lambda i: (0, 0)))
        args.append(w1n)
        args.append(b1n)
        out_specs.append(pl.BlockSpec((NC, _NBLK, CW), lambda i: (0, i, 0)))
        out_shape.append(jax.ShapeDtypeStruct((NC, N, CW), jnp.float32))
    res = pl.pallas_call(
        _dense_body_maker(l, has_next),
        grid=(N // _NBLK,),
        in_specs=in_specs,
        out_specs=out_specs,
        out_shape=out_shape,
    )(*args)
    return res if has_next else (res[0], None)


def _readout_body(h_ref, batch_ref, gw, gb, aw, p1w, p1b, pn, p2w, p2b,
                  out_ref):
    h = h_ref[...]
    z = jnp.dot(h, gw[...], preferred_element_type=jnp.float32) + gb[...]
    awr = aw[...]
    att = jnp.sum(z * awr, axis=1, keepdims=True)
    cg = jnp.sum(gb[...] * awr)
    b2 = (batch_ref[...] == lax.broadcasted_iota(jnp.int32, (N, G), 1))
    b2 = b2.astype(jnp.float32)
    counts = jnp.sum(b2, axis=0, keepdims=True)
    m = jnp.max(counts)
    neg = jnp.float32(-1e30)
    segmax = jnp.max(jnp.where(b2 > 0, att, neg), axis=0, keepdims=True)
    ghost = jnp.where(counts < m, cg, neg)
    maxg = jnp.maximum(segmax, ghost)
    maxpn = jnp.sum(b2 * maxg, axis=1, keepdims=True)
    w = jnp.exp(att - maxpn)
    denom = (jnp.sum(b2 * w, axis=0, keepdims=True)
             + (m - counts) * jnp.exp(ghost - maxg))
    denom_pn = jnp.sum(b2 * denom, axis=1, keepdims=True)
    bz = b2 * (w / denom_pn)
    hg = lax.dot_general(bz, z, (((0,), (0,)), ((), ())),
                         preferred_element_type=jnp.float32)
    zg = jnp.dot(hg, p1w[...], preferred_element_type=jnp.float32) + p1b[...]
    zg = zg * lax.rsqrt(jnp.mean(zg * zg, axis=-1, keepdims=True) + 1e-6)
    zg = zg * pn[...]
    zg = jnp.maximum(zg, 0.0)
    out_ref[...] = jnp.dot(zg, p2w[...],
                           preferred_element_type=jnp.float32) + p2b[...]


def _readout(h, batch2d, gw, gb, aw, p1w, p1b, pn, p2w, p2b):
    pdim = p2w.shape[1]
    return pl.pallas_call(
        _readout_body,
        out_shape=jax.ShapeDtypeStruct((G, pdim), jnp.float32),
        compiler_params=pltpu.CompilerParams(
            vmem_limit_bytes=100 * 1024 * 1024),
    )(h, batch2d, gw, gb, aw, p1w, p1b, pn, p2w, p2b)


def kernel(x, edge_index, edge_attr, batch, params):
    src = edge_index[0]
    dst = edge_index[1]
    convs = params["convs"]
    ffns = params["ffns"]
    norms = params["norms"]
    nlayer = len(ffns)

    w1a, b1, w1b, wut, w2u, b2u, bu = [], [], [], [], [], [], []
    for l in range(nlayer):
        cur = x.shape[1] if l == 0 else H
        m1w = convs[l]["m1"]["w"]
        w1a.append(m1w[:cur])
        b1.append(convs[l]["m1"]["b"].reshape(1, H))
        w1b.append(m1w[cur:cur + EF].reshape(EF, NC, CW).transpose(1, 0, 2))
        uw = convs[l]["u"]["w"]
        wub = uw[cur:]
        wut.append(uw[:cur])
        w2u.append(convs[l]["m2"]["w"] @ wub)
        b2u.append((convs[l]["m2"]["b"] @ wub).reshape(1, H))
        bu.append(convs[l]["u"]["b"].reshape(1, H))
    w1bs = jnp.stack(w1b)

    eaw = _eaw_kernel(edge_attr, w1bs)
    hw = _proj0(x, w1a[0], b1[0])

    srcp = jnp.concatenate([src, jnp.zeros((2 * BE,), src.dtype)])
    dstp = jnp.concatenate([dst, jnp.zeros((2 * BE,), dst.dtype)])
    gidx = _gidx(srcp.reshape(40, E2 // 40)).reshape(NC * E2)

    h = x
    deg16 = _sc_deg(dst).reshape(2, N, 16)
    for l in range(nlayer):
        eaw_l = eaw[l].reshape(NC * E, CW)
        agg = _sc_layer(hw.reshape(NC * N, CW), eaw_l, gidx, dstp)
        agg4 = agg.reshape(NC, N, CW)
        last = l == nlayer - 1
        h, hwn = _dense(
            l, h, agg4, deg16,
            wut[l], w2u[l], b2u[l], bu[l],
            norms[l].reshape(1, H),
            ffns[l]["w"], ffns[l]["b"].reshape(1, H),
            None if last else w1a[l + 1],
            None if last else b1[l + 1],
        )
        if not last:
            hw = hwn

    return _readout(
        h, batch.reshape(N, 1),
        params["gap"]["w"], params["gap"]["b"].reshape(1, H),
        params["att"]["w"].reshape(1, H),
        params["p1"]["w"], params["p1"]["b"].reshape(1, -1),
        params["pn"].reshape(1, -1),
        params["p2"]["w"], params["p2"]["b"].reshape(1, -1),
    )
